# Initial kernel scaffold; baseline (speedup 1.0000x reference)
import jax
import jax.numpy as jnp
from jax.experimental import pallas as pl


def kernel(state, W, b):
    raise NotImplementedError("write your pallas kernel here")


# speedup vs baseline: 1.2610x; 1.2610x over previous
import functools

import jax
import jax.numpy as jnp
from jax.experimental import pallas as pl
from jax.experimental.pallas import tpu as pltpu

_K = 10
_VB = 2048
_PAD = 128
_NEG = float("-inf")
_BIG_IDX = 2 ** 30


def _topk_kernel(state_ref, w_ref, b_ref, out_ref, svals_ref, sidx_ref,
                 *, nblocks, v_total):
    j = pl.program_id(0)

    @pl.when(j == 0)
    def _init():
        svals_ref[...] = jnp.full(svals_ref.shape, _NEG, jnp.float32)
        sidx_ref[...] = jnp.full(sidx_ref.shape, _BIG_IDX, jnp.int32)

    logits = jnp.dot(state_ref[...], w_ref[...],
                     preferred_element_type=jnp.float32)
    logits = logits + b_ref[...]
    col = jax.lax.broadcasted_iota(jnp.int32, logits.shape, 1) + j * _VB
    logits = jnp.where(col < v_total, logits, _NEG)

    row_max = jnp.max(logits, axis=1)
    thresh = svals_ref[:, _K - 1]
    need_merge = jnp.any(row_max > thresh)

    @pl.when(need_merge)
    def _merge():
        work_v = jnp.concatenate([svals_ref[...], logits], axis=1)
        work_i = jnp.concatenate([sidx_ref[...], col], axis=1)
        vals = []
        idxs = []
        for _ in range(_K):
            m = jnp.max(work_v, axis=1, keepdims=True)
            is_max = work_v == m
            sel = jnp.min(jnp.where(is_max, work_i, _BIG_IDX),
                          axis=1, keepdims=True)
            vals.append(m)
            idxs.append(sel)
            work_v = jnp.where(work_i == sel, _NEG, work_v)
        svals_ref[:, 0:_K] = jnp.concatenate(vals, axis=1)
        sidx_ref[:, 0:_K] = jnp.concatenate(idxs, axis=1)

    @pl.when(j == nblocks - 1)
    def _out():
        out_ref[...] = sidx_ref[:, 0:_K]


@jax.jit
def kernel(state, W, b):
    batch, hidden = state.shape
    v_total = W.shape[1]
    nblocks = pl.cdiv(v_total, _VB)
    b2 = b.reshape(1, v_total)
    return pl.pallas_call(
        functools.partial(_topk_kernel, nblocks=nblocks, v_total=v_total),
        grid=(nblocks,),
        in_specs=[
            pl.BlockSpec((batch, hidden), lambda j: (0, 0)),
            pl.BlockSpec((hidden, _VB), lambda j: (0, j)),
            pl.BlockSpec((1, _VB), lambda j: (0, j)),
        ],
        out_specs=pl.BlockSpec((batch, _K), lambda j: (0, 0)),
        out_shape=jax.ShapeDtypeStruct((batch, _K), jnp.int32),
        scratch_shapes=[
            pltpu.VMEM((batch, _PAD), jnp.float32),
            pltpu.VMEM((batch, _PAD), jnp.int32),
        ],
    )(state, W, b2)
